# baseline (speedup 1.0000x reference)
import jax
import jax.numpy as jnp
from jax.experimental import pallas as pl
from jax.experimental.pallas import tpu as pltpu


def _body(f_ref, aux_ref, W1_ref, W2_ref, Wp_ref, b1_ref, b2_ref, bp_ref,
          o_ref, M_s, G_s, c_s):
    Bblk, D = f_ref.shape
    C = 24
    i = pl.program_id(0)

    @pl.when(i == 0)
    def _collapse():
        hp = jax.lax.Precision.HIGHEST

        def dot(a, b):
            return jax.lax.dot(a, b, precision=hp,
                               preferred_element_type=jnp.float32)

        zcol = jnp.zeros((D, 1), jnp.float32)
        M_s[...] = jnp.concatenate([
            dot(W1_ref[0:D, :], W2_ref[:, 0:1]),
            dot(W1_ref[D:2 * D, :], W2_ref[:, 1:7]), zcol,
            dot(W1_ref[2 * D:3 * D, :], W2_ref[:, 7:8]),
            dot(W1_ref[3 * D:4 * D, :], W2_ref[:, 8:14]), zcol,
            dot(W1_ref[4 * D:5 * D, :], W2_ref[:, 14:15]),
            dot(W1_ref[5 * D + 32:6 * D + 32, :], W2_ref[:, 15:21]), zcol,
        ], axis=1).astype(jnp.bfloat16)
        Tc = dot(W1_ref[5 * D:5 * D + 32, :], W2_ref[:, 14:15])
        Ta = dot(W1_ref[6 * D + 32:6 * D + 64, :], W2_ref[:, 15:21])
        Gblk = jnp.concatenate([
            jnp.zeros((2, 16), jnp.float32),
            dot(Wp_ref[0:2, :], Tc), dot(Wp_ref[2:4, :], Ta),
            jnp.zeros((2, 1), jnp.float32),
        ], axis=1)
        G_s[...] = jnp.concatenate(
            [Gblk, jnp.zeros((6, C), jnp.float32)], axis=0)
        z1 = jnp.zeros((1, 1), jnp.float32)
        crow = jnp.concatenate([
            dot(b1_ref[0:1, :], W2_ref[:, 0:1]) + b2_ref[:, 0:1],
            dot(b1_ref[1:2, :], W2_ref[:, 1:7]) + b2_ref[:, 1:7], z1,
            dot(b1_ref[2:3, :], W2_ref[:, 7:8]) + b2_ref[:, 7:8],
            dot(b1_ref[3:4, :], W2_ref[:, 8:14]) + b2_ref[:, 8:14], z1,
            dot(b1_ref[4:5, :], W2_ref[:, 14:15]) + b2_ref[:, 14:15]
            + dot(bp_ref[0:1, :], Tc),
            dot(b1_ref[5:6, :], W2_ref[:, 15:21]) + b2_ref[:, 15:21]
            + dot(bp_ref[1:2, :], Ta),
            z1,
        ], axis=1)
        c_s[...] = jnp.concatenate(
            [crow, jnp.zeros((7, C), jnp.float32)], axis=0)

    tcol = aux_ref[:, 0:1]
    p0col = aux_ref[:, 1:2]
    p1col = aux_ref[:, 2:3]

    Y = jnp.dot(f_ref[...].astype(jnp.bfloat16), M_s[...],
                preferred_element_type=jnp.float32)
    Y = (Y + p0col * G_s[0:1, :] + p1col * G_s[1:2, :] + c_s[0:1, :])
    grp = (jax.lax.broadcasted_iota(jnp.int32, (1, C), 1) // 8)
    mask = (grp.astype(jnp.float32) == tcol).astype(jnp.float32)
    rowmod = jax.lax.broadcasted_iota(jnp.int32, (C, 7), 0) % 8
    colj = jax.lax.broadcasted_iota(jnp.int32, (C, 7), 1)
    sel = (rowmod == colj).astype(jnp.bfloat16)
    o_ref[...] = jnp.dot((Y * mask).astype(jnp.bfloat16), sel,
                         preferred_element_type=jnp.float32)


def kernel(features, task_id, pointgoal_with_gps_compass,
           a0W1, a0b1, a0W2, a0b2,
           a1W1, a1b1, a1W2, a1b2,
           a2Wp, a2bp, a2W1, a2b1, a2W2, a2b2,
           c0W1, c0b1, c0W2, c0b2,
           c1W1, c1b1, c1W2, c1b2,
           c2Wp, c2bp, c2W1, c2b1, c2W2, c2b2):
    B, D = features.shape
    Bblk = 2048
    nb = B // Bblk

    aux = jnp.concatenate([task_id, pointgoal_with_gps_compass], axis=1)
    r = lambda x: x.reshape(1, -1)
    W1 = jnp.concatenate([c0W1, a0W1, c1W1, a1W1, c2W1, a2W1], axis=0)
    W2 = jnp.concatenate([c0W2, a0W2, c1W2, a1W2, c2W2, a2W2], axis=1)
    Wp = jnp.concatenate([c2Wp, a2Wp], axis=0)
    b1 = jnp.concatenate([r(c0b1), r(a0b1), r(c1b1), r(a1b1),
                          r(c2b1), r(a2b1)], axis=0)
    b2 = jnp.concatenate([r(c0b2), r(a0b2), r(c1b2), r(a1b2),
                          r(c2b2), r(a2b2)], axis=1)
    bp = jnp.concatenate([r(c2bp), r(a2bp)], axis=0)

    full_spec = lambda a: pl.BlockSpec(a.shape, lambda i: (0,) * a.ndim)

    return pl.pallas_call(
        _body,
        grid=(nb,),
        in_specs=[pl.BlockSpec((Bblk, D), lambda i: (i, 0)),
                  pl.BlockSpec((Bblk, 3), lambda i: (i, 0)),
                  full_spec(W1), full_spec(W2), full_spec(Wp),
                  full_spec(b1), full_spec(b2), full_spec(bp)],
        out_specs=pl.BlockSpec((Bblk, 7), lambda i: (i, 0)),
        out_shape=jax.ShapeDtypeStruct((B, 7), jnp.float32),
        scratch_shapes=[pltpu.VMEM((D, 24), jnp.bfloat16),
                        pltpu.VMEM((8, 24), jnp.float32),
                        pltpu.VMEM((8, 24), jnp.float32)],
        compiler_params=pltpu.CompilerParams(
            dimension_semantics=("arbitrary",)),
    )(features, aux, W1, W2, Wp, b1, b2, bp)


# speedup vs baseline: 1.0775x; 1.0775x over previous
import jax
import jax.numpy as jnp
from jax.experimental import pallas as pl
from jax.experimental.pallas import tpu as pltpu


def _body(f_ref, aux_ref, c0W1_ref, a0W1_ref, c1W1_ref, a1W1_ref,
          c2W1_ref, a2W1_ref, W2_ref, Wp_ref, b1_ref, b2_ref, bp_ref,
          o_ref, M_s, G_s, c_s):
    Bblk, D = f_ref.shape
    C = 24
    i = pl.program_id(0)

    @pl.when(i == 0)
    def _collapse():
        hp = jax.lax.Precision.HIGHEST

        def dot(a, b):
            return jax.lax.dot(a, b, precision=hp,
                               preferred_element_type=jnp.float32)

        zcol = jnp.zeros((D, 1), jnp.float32)
        M_s[...] = jnp.concatenate([
            dot(c0W1_ref[...], W2_ref[:, 0:1]),
            dot(a0W1_ref[...], W2_ref[:, 1:7]), zcol,
            dot(c1W1_ref[...], W2_ref[:, 7:8]),
            dot(a1W1_ref[...], W2_ref[:, 8:14]), zcol,
            dot(c2W1_ref[0:D, :], W2_ref[:, 14:15]),
            dot(a2W1_ref[0:D, :], W2_ref[:, 15:21]), zcol,
        ], axis=1).astype(jnp.bfloat16)
        Tc = dot(c2W1_ref[D:, :], W2_ref[:, 14:15])
        Ta = dot(a2W1_ref[D:, :], W2_ref[:, 15:21])
        Gblk = jnp.concatenate([
            jnp.zeros((2, 16), jnp.float32),
            dot(Wp_ref[0:2, :], Tc), dot(Wp_ref[2:4, :], Ta),
            jnp.zeros((2, 1), jnp.float32),
        ], axis=1)
        G_s[...] = jnp.concatenate(
            [Gblk, jnp.zeros((6, C), jnp.float32)], axis=0)
        z1 = jnp.zeros((1, 1), jnp.float32)
        crow = jnp.concatenate([
            dot(b1_ref[0:1, :], W2_ref[:, 0:1]) + b2_ref[:, 0:1],
            dot(b1_ref[1:2, :], W2_ref[:, 1:7]) + b2_ref[:, 1:7], z1,
            dot(b1_ref[2:3, :], W2_ref[:, 7:8]) + b2_ref[:, 7:8],
            dot(b1_ref[3:4, :], W2_ref[:, 8:14]) + b2_ref[:, 8:14], z1,
            dot(b1_ref[4:5, :], W2_ref[:, 14:15]) + b2_ref[:, 14:15]
            + dot(bp_ref[0:1, :], Tc),
            dot(b1_ref[5:6, :], W2_ref[:, 15:21]) + b2_ref[:, 15:21]
            + dot(bp_ref[1:2, :], Ta),
            z1,
        ], axis=1)
        c_s[...] = jnp.concatenate(
            [crow, jnp.zeros((7, C), jnp.float32)], axis=0)

    tcol = aux_ref[:, 0:1]
    p0col = aux_ref[:, 1:2]
    p1col = aux_ref[:, 2:3]

    Y = jnp.dot(f_ref[...].astype(jnp.bfloat16), M_s[...],
                preferred_element_type=jnp.float32)
    Y = (Y + p0col * G_s[0:1, :] + p1col * G_s[1:2, :] + c_s[0:1, :])
    grp = (jax.lax.broadcasted_iota(jnp.int32, (1, C), 1) // 8)
    mask = (grp.astype(jnp.float32) == tcol).astype(jnp.float32)
    rowmod = jax.lax.broadcasted_iota(jnp.int32, (C, 7), 0) % 8
    colj = jax.lax.broadcasted_iota(jnp.int32, (C, 7), 1)
    sel = (rowmod == colj).astype(jnp.bfloat16)
    o_ref[...] = jnp.dot((Y * mask).astype(jnp.bfloat16), sel,
                         preferred_element_type=jnp.float32)


def kernel(features, task_id, pointgoal_with_gps_compass,
           a0W1, a0b1, a0W2, a0b2,
           a1W1, a1b1, a1W2, a1b2,
           a2Wp, a2bp, a2W1, a2b1, a2W2, a2b2,
           c0W1, c0b1, c0W2, c0b2,
           c1W1, c1b1, c1W2, c1b2,
           c2Wp, c2bp, c2W1, c2b1, c2W2, c2b2):
    B, D = features.shape
    Bblk = 2048
    nb = B // Bblk

    aux = jnp.concatenate([task_id, pointgoal_with_gps_compass], axis=1)
    r = lambda x: x.reshape(1, -1)
    W2 = jnp.concatenate([c0W2, a0W2, c1W2, a1W2, c2W2, a2W2], axis=1)
    Wp = jnp.concatenate([c2Wp, a2Wp], axis=0)
    b1 = jnp.concatenate([r(c0b1), r(a0b1), r(c1b1), r(a1b1),
                          r(c2b1), r(a2b1)], axis=0)
    b2 = jnp.concatenate([r(c0b2), r(a0b2), r(c1b2), r(a1b2),
                          r(c2b2), r(a2b2)], axis=1)
    bp = jnp.concatenate([r(c2bp), r(a2bp)], axis=0)

    full_spec = lambda a: pl.BlockSpec(a.shape, lambda i: (0,) * a.ndim)

    return pl.pallas_call(
        _body,
        grid=(nb,),
        in_specs=[pl.BlockSpec((Bblk, D), lambda i: (i, 0)),
                  pl.BlockSpec((Bblk, 3), lambda i: (i, 0)),
                  full_spec(c0W1), full_spec(a0W1), full_spec(c1W1),
                  full_spec(a1W1), full_spec(c2W1), full_spec(a2W1),
                  full_spec(W2), full_spec(Wp),
                  full_spec(b1), full_spec(b2), full_spec(bp)],
        out_specs=pl.BlockSpec((Bblk, 7), lambda i: (i, 0)),
        out_shape=jax.ShapeDtypeStruct((B, 7), jnp.float32),
        scratch_shapes=[pltpu.VMEM((D, 24), jnp.bfloat16),
                        pltpu.VMEM((8, 24), jnp.float32),
                        pltpu.VMEM((8, 24), jnp.float32)],
        compiler_params=pltpu.CompilerParams(
            dimension_semantics=("arbitrary",)),
    )(features, aux, c0W1, a0W1, c1W1, a1W1, c2W1, a2W1,
      W2, Wp, b1, b2, bp)


# speedup vs baseline: 1.1060x; 1.0265x over previous
import jax
import jax.numpy as jnp
from jax.experimental import pallas as pl
from jax.experimental.pallas import tpu as pltpu


def _body(f_ref, aux_ref, c0W1_ref, a0W1_ref, c1W1_ref, a1W1_ref,
          c2W1_ref, a2W1_ref, W2_ref, Wp_ref, b1_ref, b2_ref, bp_ref,
          o_ref, M_s, G_s, c_s):
    Bblk, D = f_ref.shape
    C = 24
    i = pl.program_id(0)

    @pl.when(i == 0)
    def _collapse():
        hp = jax.lax.Precision.HIGHEST

        def dot(a, b):
            return jax.lax.dot(a, b, precision=hp,
                               preferred_element_type=jnp.float32)

        zcol = jnp.zeros((D, 1), jnp.float32)
        M_s[...] = jnp.concatenate([
            dot(c0W1_ref[...], W2_ref[:, 0:1]),
            dot(a0W1_ref[...], W2_ref[:, 1:7]), zcol,
            dot(c1W1_ref[...], W2_ref[:, 7:8]),
            dot(a1W1_ref[...], W2_ref[:, 8:14]), zcol,
            dot(c2W1_ref[0:D, :], W2_ref[:, 14:15]),
            dot(a2W1_ref[0:D, :], W2_ref[:, 15:21]), zcol,
        ], axis=1).astype(jnp.bfloat16)
        Tc = dot(c2W1_ref[D:, :], W2_ref[:, 14:15])
        Ta = dot(a2W1_ref[D:, :], W2_ref[:, 15:21])
        Gblk = jnp.concatenate([
            jnp.zeros((2, 16), jnp.float32),
            dot(Wp_ref[0:2, :], Tc), dot(Wp_ref[2:4, :], Ta),
            jnp.zeros((2, 1), jnp.float32),
        ], axis=1)
        G_s[...] = jnp.concatenate(
            [Gblk, jnp.zeros((6, C), jnp.float32)], axis=0)
        z1 = jnp.zeros((1, 1), jnp.float32)
        crow = jnp.concatenate([
            dot(b1_ref[0:1, :], W2_ref[:, 0:1]) + b2_ref[:, 0:1],
            dot(b1_ref[1:2, :], W2_ref[:, 1:7]) + b2_ref[:, 1:7], z1,
            dot(b1_ref[2:3, :], W2_ref[:, 7:8]) + b2_ref[:, 7:8],
            dot(b1_ref[3:4, :], W2_ref[:, 8:14]) + b2_ref[:, 8:14], z1,
            dot(b1_ref[4:5, :], W2_ref[:, 14:15]) + b2_ref[:, 14:15]
            + dot(bp_ref[0:1, :], Tc),
            dot(b1_ref[5:6, :], W2_ref[:, 15:21]) + b2_ref[:, 15:21]
            + dot(bp_ref[1:2, :], Ta),
            z1,
        ], axis=1)
        c_s[...] = jnp.concatenate(
            [crow, jnp.zeros((7, C), jnp.float32)], axis=0)

    tcol = aux_ref[:, 0:1]
    p0col = aux_ref[:, 1:2]
    p1col = aux_ref[:, 2:3]

    Y = jnp.dot(f_ref[...].astype(jnp.bfloat16), M_s[...],
                preferred_element_type=jnp.float32)
    Y = (Y + p0col * G_s[0:1, :] + p1col * G_s[1:2, :] + c_s[0:1, :])
    grp = (jax.lax.broadcasted_iota(jnp.int32, (1, C), 1) // 8)
    mask = (grp.astype(jnp.float32) == tcol).astype(jnp.float32)
    rowmod = jax.lax.broadcasted_iota(jnp.int32, (C, 7), 0) % 8
    colj = jax.lax.broadcasted_iota(jnp.int32, (C, 7), 1)
    sel = (rowmod == colj).astype(jnp.bfloat16)
    o_ref[...] = jnp.dot((Y * mask).astype(jnp.bfloat16), sel,
                         preferred_element_type=jnp.float32)


def kernel(features, task_id, pointgoal_with_gps_compass,
           a0W1, a0b1, a0W2, a0b2,
           a1W1, a1b1, a1W2, a1b2,
           a2Wp, a2bp, a2W1, a2b1, a2W2, a2b2,
           c0W1, c0b1, c0W2, c0b2,
           c1W1, c1b1, c1W2, c1b2,
           c2Wp, c2bp, c2W1, c2b1, c2W2, c2b2):
    B, D = features.shape
    Bblk = 4096
    nb = B // Bblk

    aux = jnp.concatenate([task_id, pointgoal_with_gps_compass], axis=1)
    r = lambda x: x.reshape(1, -1)
    W2 = jnp.concatenate([c0W2, a0W2, c1W2, a1W2, c2W2, a2W2], axis=1)
    Wp = jnp.concatenate([c2Wp, a2Wp], axis=0)
    b1 = jnp.concatenate([r(c0b1), r(a0b1), r(c1b1), r(a1b1),
                          r(c2b1), r(a2b1)], axis=0)
    b2 = jnp.concatenate([r(c0b2), r(a0b2), r(c1b2), r(a1b2),
                          r(c2b2), r(a2b2)], axis=1)
    bp = jnp.concatenate([r(c2bp), r(a2bp)], axis=0)

    full_spec = lambda a: pl.BlockSpec(a.shape, lambda i: (0,) * a.ndim)

    return pl.pallas_call(
        _body,
        grid=(nb,),
        in_specs=[pl.BlockSpec((Bblk, D), lambda i: (i, 0)),
                  pl.BlockSpec((Bblk, 3), lambda i: (i, 0)),
                  full_spec(c0W1), full_spec(a0W1), full_spec(c1W1),
                  full_spec(a1W1), full_spec(c2W1), full_spec(a2W1),
                  full_spec(W2), full_spec(Wp),
                  full_spec(b1), full_spec(b2), full_spec(bp)],
        out_specs=pl.BlockSpec((Bblk, 7), lambda i: (i, 0)),
        out_shape=jax.ShapeDtypeStruct((B, 7), jnp.float32),
        scratch_shapes=[pltpu.VMEM((D, 24), jnp.bfloat16),
                        pltpu.VMEM((8, 24), jnp.float32),
                        pltpu.VMEM((8, 24), jnp.float32)],
        compiler_params=pltpu.CompilerParams(
            dimension_semantics=("arbitrary",)),
    )(features, aux, c0W1, a0W1, c1W1, a1W1, c2W1, a2W1,
      W2, Wp, b1, b2, bp)
